# baseline (speedup 1.0000x reference)
import functools
import math

import jax
import jax.numpy as jnp
from jax.experimental import pallas as pl
from jax.experimental.pallas import tpu as pltpu

S = 2048
D = 1024
H = 16
DK = 64
F = 2048
E = 8
V = 32000
SQRTD = math.sqrt(D)


def _rope_coefs():
    inv_freq = 1.0 / (10000.0 ** (jnp.arange(0, D, 2, dtype=jnp.float32) / D))
    t = jnp.arange(S, dtype=jnp.float32)
    si = t[:, None] * inv_freq[None, :]
    sin = jnp.sin(si)
    cos = jnp.cos(si)
    c = jnp.repeat(cos, 2, axis=1) * SQRTD
    dmask = (jnp.arange(D) % 2 == 0)
    a = jnp.where(dmask[None, :], -jnp.repeat(sin, 2, axis=1), 0.0) * SQRTD
    b = jnp.where(dmask[None, :], 0.0, jnp.repeat(sin, 2, axis=1)) * SQRTD
    return c, a, b


_NR = 8


def _embed_rope_kernel(ids_ref, *refs):
    emb_rows = refs[:_NR]
    c_ref, a_ref, b_ref, o_ref = refs[_NR:]
    for j in range(_NR):
        row = emb_rows[j][0]
        shl = jnp.concatenate([row[:, 1:], row[:, :1]], axis=1)
        shr = jnp.concatenate([row[:, -1:], row[:, :-1]], axis=1)
        o_ref[j:j + 1, :] = (row * c_ref[j:j + 1, :]
                             + shl * a_ref[j:j + 1, :]
                             + shr * b_ref[j:j + 1, :])


def _embed_rope(ids, emb, c, a, b):
    grid = (S // _NR,)
    emb = emb.reshape(V, 1, D)
    emb_specs = [
        pl.BlockSpec((1, 1, D), functools.partial(
            lambda i, ids, j: (ids[i * _NR + j], 0, 0), j=j))
        for j in range(_NR)
    ]
    coef_spec = pl.BlockSpec((_NR, D), lambda i, ids: (i, 0))
    spec = pltpu.PrefetchScalarGridSpec(
        num_scalar_prefetch=1,
        grid=grid,
        in_specs=emb_specs + [coef_spec] * 3,
        out_specs=pl.BlockSpec((_NR, D), lambda i, ids: (i, 0)),
    )
    return pl.pallas_call(
        _embed_rope_kernel,
        grid_spec=spec,
        out_shape=jax.ShapeDtypeStruct((S, D), jnp.float32),
    )(ids, *([emb] * _NR), c, a, b)


def _matmul_kernel(x_ref, w_ref, b_ref, o_ref):
    o_ref[...] = (jnp.dot(x_ref[...], w_ref[...],
                          preferred_element_type=jnp.float32)
                  + b_ref[...])


def _matmul(x, w, b, bn):
    m, k = x.shape
    n = w.shape[1]
    assert n % bn == 0
    return pl.pallas_call(
        _matmul_kernel,
        grid=(n // bn,),
        in_specs=[
            pl.BlockSpec((m, k), lambda j: (0, 0)),
            pl.BlockSpec((k, bn), lambda j: (0, j)),
            pl.BlockSpec((1, bn), lambda j: (0, j)),
        ],
        out_specs=pl.BlockSpec((m, bn), lambda j: (0, j)),
        out_shape=jax.ShapeDtypeStruct((m, n), jnp.float32),
    )(x, w, b.reshape(1, n))


_BQ = 256


def _qkv_proj_kernel(x_ref, w_ref, b_ref, o_ref):
    o_ref[0] = (jnp.dot(x_ref[...], w_ref[0],
                        preferred_element_type=jnp.float32) + b_ref[0])


def _qkv_proj(x, w, b):
    w3 = w.reshape(D, H, DK).transpose(1, 0, 2)
    b3 = b.reshape(H, 1, DK)
    return pl.pallas_call(
        _qkv_proj_kernel,
        grid=(H,),
        in_specs=[
            pl.BlockSpec((S, D), lambda h: (0, 0)),
            pl.BlockSpec((1, D, DK), lambda h: (h, 0, 0)),
            pl.BlockSpec((1, 1, DK), lambda h: (h, 0, 0)),
        ],
        out_specs=pl.BlockSpec((1, S, DK), lambda h: (h, 0, 0)),
        out_shape=jax.ShapeDtypeStruct((H, S, DK), jnp.float32),
    )(x, w3, b3)


def _attn_kernel(q_ref, k_ref, v_ref, o_ref):
    s = jax.lax.dot_general(q_ref[0], k_ref[0],
                            (((1,), (1,)), ((), ())),
                            preferred_element_type=jnp.float32)
    s = s * (1.0 / math.sqrt(DK))
    mx = jnp.max(s, axis=-1, keepdims=True)
    p = jnp.exp(s - mx)
    o = jnp.dot(p, v_ref[0], preferred_element_type=jnp.float32)
    o_ref[0] = o / jnp.sum(p, axis=-1, keepdims=True)


def _attention(q, k, v):
    return pl.pallas_call(
        _attn_kernel,
        grid=(H, S // _BQ),
        in_specs=[
            pl.BlockSpec((1, _BQ, DK), lambda h, i: (h, i, 0)),
            pl.BlockSpec((1, S, DK), lambda h, i: (h, 0, 0)),
            pl.BlockSpec((1, S, DK), lambda h, i: (h, 0, 0)),
        ],
        out_specs=pl.BlockSpec((1, _BQ, DK), lambda h, i: (h, i, 0)),
        out_shape=jax.ShapeDtypeStruct((H, S, DK), jnp.float32),
    )(q, k, v)


def _o_proj_kernel(a_ref, w_ref, b_ref, o_ref):
    h = pl.program_id(0)
    part = jnp.dot(a_ref[0], w_ref[0], preferred_element_type=jnp.float32)

    @pl.when(h == 0)
    def _init():
        o_ref[...] = part + b_ref[...]

    @pl.when(h != 0)
    def _acc():
        o_ref[...] += part


def _o_proj(a, w, b):
    w3 = w.reshape(H, DK, D)
    return pl.pallas_call(
        _o_proj_kernel,
        grid=(H,),
        in_specs=[
            pl.BlockSpec((1, S, DK), lambda h: (h, 0, 0)),
            pl.BlockSpec((1, DK, D), lambda h: (h, 0, 0)),
            pl.BlockSpec((1, D), lambda h: (0, 0)),
        ],
        out_specs=pl.BlockSpec((S, D), lambda h: (0, 0)),
        out_shape=jax.ShapeDtypeStruct((S, D), jnp.float32),
    )(a, w3, b.reshape(1, D))


def _mha(p, q, k, v):
    Q = _qkv_proj(q, p['Wq'], p['bq'])
    K = _qkv_proj(k, p['Wk'], p['bk'])
    Vv = _qkv_proj(v, p['Wv'], p['bv'])
    o = _attention(Q, K, Vv)
    return _o_proj(o, p['Wo'], p['bo'])


def _add_ln_kernel(x_ref, d_ref, g_ref, b_ref, o_ref):
    y = x_ref[...] + d_ref[...]
    mu = jnp.mean(y, axis=-1, keepdims=True)
    yc = y - mu
    var = jnp.mean(yc * yc, axis=-1, keepdims=True)
    o_ref[...] = yc * jax.lax.rsqrt(var + 1e-5) * g_ref[...] + b_ref[...]


def _add_ln(x, delta, lnp):
    bs = 256
    return pl.pallas_call(
        _add_ln_kernel,
        grid=(S // bs,),
        in_specs=[
            pl.BlockSpec((bs, D), lambda i: (i, 0)),
            pl.BlockSpec((bs, D), lambda i: (i, 0)),
            pl.BlockSpec((1, D), lambda i: (0, 0)),
            pl.BlockSpec((1, D), lambda i: (0, 0)),
        ],
        out_specs=pl.BlockSpec((bs, D), lambda i: (i, 0)),
        out_shape=jax.ShapeDtypeStruct((S, D), jnp.float32),
    )(x, delta, lnp['g'].reshape(1, D), lnp['b'].reshape(1, D))


def _gate_kernel(x_ref, w_ref, b_ref, wg_ref):
    s = jnp.dot(x_ref[...], w_ref[...],
                preferred_element_type=jnp.float32) + b_ref[...]
    cols = jax.lax.broadcasted_iota(jnp.int32, s.shape, 1)
    m1 = jnp.max(s, axis=-1, keepdims=True)
    i1 = jnp.min(jnp.where(s == m1, cols, E), axis=-1, keepdims=True)
    s2 = jnp.where(cols == i1, -jnp.inf, s)
    m2 = jnp.max(s2, axis=-1, keepdims=True)
    i2 = jnp.min(jnp.where(s2 == m2, cols, E), axis=-1, keepdims=True)
    ex = jnp.exp(m2 - m1)
    w1 = 1.0 / (1.0 + ex)
    w2 = 1.0 - w1
    wg_ref[...] = jnp.where(cols == i1, w1,
                            jnp.where(cols == i2, w2, 0.0))


def _gate(x, gw, gb):
    bs = 256
    return pl.pallas_call(
        _gate_kernel,
        grid=(S // bs,),
        in_specs=[
            pl.BlockSpec((bs, D), lambda i: (i, 0)),
            pl.BlockSpec((D, E), lambda i: (0, 0)),
            pl.BlockSpec((1, E), lambda i: (0, 0)),
        ],
        out_specs=pl.BlockSpec((bs, E), lambda i: (i, 0)),
        out_shape=jax.ShapeDtypeStruct((S, E), jnp.float32),
    )(x, gw, gb.reshape(1, E))


_BME = 256
_BF = 1024


def _moe_dense_kernel(x_ref, wg_ref, w1_ref, b1_ref, w2_ref, b2_ref, o_ref):
    e = pl.program_id(1)
    f = pl.program_id(2)
    h = (jnp.dot(x_ref[...], w1_ref[0], preferred_element_type=jnp.float32)
         + b1_ref[0])
    gelu = 0.5 * h * (1.0 + jax.lax.erf(h * (1.0 / math.sqrt(2.0))))
    silu = h * jax.nn.sigmoid(h)
    h = jnp.where(e % 2 == 0, gelu, silu)
    part = jnp.dot(h, w2_ref[0], preferred_element_type=jnp.float32)
    onehot = (jax.lax.broadcasted_iota(jnp.int32, (E, 1), 0) == e
              ).astype(jnp.float32)
    w = jnp.dot(wg_ref[...], onehot, preferred_element_type=jnp.float32)
    contrib = w * part
    contrib = contrib + jnp.where(f == 0, 1.0, 0.0) * (w * b2_ref[0])

    @pl.when(jnp.logical_and(e == 0, f == 0))
    def _init():
        o_ref[...] = contrib

    @pl.when(jnp.logical_or(e != 0, f != 0))
    def _acc():
        o_ref[...] += contrib


def _moe(p, x):
    wg = _gate(x, p['gate_W'], p['gate_b'])
    return pl.pallas_call(
        _moe_dense_kernel,
        grid=(S // _BME, E, F // _BF),
        in_specs=[
            pl.BlockSpec((_BME, D), lambda m, e, f: (m, 0)),
            pl.BlockSpec((_BME, E), lambda m, e, f: (m, 0)),
            pl.BlockSpec((1, D, _BF), lambda m, e, f: (e, 0, f)),
            pl.BlockSpec((1, 1, _BF), lambda m, e, f: (e, 0, f)),
            pl.BlockSpec((1, _BF, D), lambda m, e, f: (e, f, 0)),
            pl.BlockSpec((1, 1, D), lambda m, e, f: (e, 0, 0)),
        ],
        out_specs=pl.BlockSpec((_BME, D), lambda m, e, f: (m, 0)),
        out_shape=jax.ShapeDtypeStruct((S, D), jnp.float32),
    )(x, wg, p['W1'], p['b1'].reshape(E, 1, F), p['W2'],
      p['b2'].reshape(E, 1, D))


def _block(p, x, enc_out=None):
    x = _add_ln(x, _mha(p['sa'], x, x, x), p['ln1'])
    if enc_out is not None:
        x = _add_ln(x, _mha(p['ca'], x, enc_out, enc_out), p['ln2'])
    return _add_ln(x, _moe(p['moe'], x), p['ln3'])


def kernel(src, tgt, params):
    src = src.reshape(-1).astype(jnp.int32)
    tgt = tgt.reshape(-1).astype(jnp.int32)
    emb = params['embedding']
    c, a, b = _rope_coefs()
    se = _embed_rope(src, emb, c, a, b)
    se = _block(params['enc'][0], se)
    te = _embed_rope(tgt, emb, c, a, b)
    te = _block(params['dec'][0], te, enc_out=se)
    logits = _matmul(te, params['out_W'], params['out_b'], 640)
    return logits.reshape(1, S, -1)
